# Initial kernel scaffold; baseline (speedup 1.0000x reference)
import jax
import jax.numpy as jnp
from jax.experimental import pallas as pl


def kernel(species12, distances, order, cutoff_radii, sr6, sr8):
    raise NotImplementedError("write your pallas kernel here")


# speedup vs baseline: 111.5549x; 111.5549x over previous
import functools

import jax
import jax.numpy as jnp
from jax import lax
from jax.experimental import pallas as pl
from jax.experimental.pallas import tpu as pltpu
from jax.experimental.pallas import tpu_sc as plsc


def _sc_geometry():
    try:
        info = plsc.get_sparse_core_info()
        return info.num_cores, info.num_subcores, info.num_lanes
    except Exception:
        return 2, 16, 16


@functools.lru_cache(maxsize=None)
def _build(P, NELEM, TAB, C):
    NC, NS, L = _sc_geometry()
    NW = NC * NS
    per_w = P // NW
    n_chunks = per_w // C
    n_vec = C // L
    mesh = plsc.VectorSubcoreMesh(core_axis_name="c", subcore_axis_name="s")

    @functools.partial(
        pl.kernel,
        out_type=jax.ShapeDtypeStruct((P,), jnp.float32),
        mesh=mesh,
        scratch_types=[
            pltpu.VMEM((TAB,), jnp.float32),
            pltpu.VMEM((L,), jnp.int32),
            pltpu.VMEM((C,), jnp.int32),
            pltpu.VMEM((C,), jnp.int32),
            pltpu.VMEM((C,), jnp.float32),
            pltpu.VMEM((C,), jnp.float32),
        ],
        compiler_params=pltpu.CompilerParams(needs_layout_passes=False),
    )
    def sc_kernel(sp_hbm, d_hbm, tab_hbm, sel_hbm, out_hbm,
                  tab_v, sel_v, s0_v, s1_v, d_v, out_v):
        wid = lax.axis_index("s") * NC + lax.axis_index("c")
        base = pl.multiple_of(wid * per_w, 8)
        pltpu.sync_copy(tab_hbm, tab_v)
        pltpu.sync_copy(sel_hbm, sel_v)
        is6 = sel_v[...] > 0

        def chunk_body(j, carry):
            off = pl.multiple_of(base + j * C, 8)
            pltpu.sync_copy(sp_hbm.at[pl.ds(off, C)], s0_v)
            pltpu.sync_copy(sp_hbm.at[pl.ds(P + off, C)], s1_v)
            pltpu.sync_copy(d_hbm.at[pl.ds(off, C)], d_v)

            def vec_body(i, carry2):
                s0 = s0_v[pl.ds(i * L, L)]
                s1 = s1_v[pl.ds(i * L, L)]
                dd = d_v[pl.ds(i * L, L)]
                idx = s0 * NELEM + s1
                g = plsc.load_gather(tab_v, [idx])
                r = 1.0 / (1.0 + dd * g)
                t = dd * r
                t2 = t * t
                t4 = t2 * t2
                t6 = t4 * t2
                t8 = t4 * t4
                r2 = r * r
                r4 = r2 * r2
                r8 = r4 * r4
                ts = jnp.where(is6, t6, t8)
                out_v[pl.ds(i * L, L)] = ts * r8
                return carry2

            lax.fori_loop(0, n_vec, vec_body, 0, unroll=4)
            pltpu.sync_copy(out_v, out_hbm.at[pl.ds(off, C)])
            return carry

        lax.fori_loop(0, n_chunks, chunk_body, 0)

    return sc_kernel


def kernel(species12, distances, order, cutoff_radii, sr6, sr8):
    P = distances.shape[0]
    NELEM = cutoff_radii.shape[0]
    is6 = order == 6
    s = jnp.where(is6, sr6, sr8)
    itab = (jnp.float32(6.0) / (s * cutoff_radii)).reshape(-1)
    TAB = itab.shape[0]
    sel = jnp.where(is6, jnp.int32(1), jnp.int32(0))
    sel_vec = jnp.full((16,), sel, dtype=jnp.int32)
    sc_kernel = _build(P, NELEM, TAB, 10000)
    return sc_kernel(species12.reshape(-1), distances, itab, sel_vec)


# speedup vs baseline: 124.5190x; 1.1162x over previous
import functools

import jax
import jax.numpy as jnp
from jax import lax
from jax.experimental import pallas as pl
from jax.experimental.pallas import tpu as pltpu
from jax.experimental.pallas import tpu_sc as plsc


def _sc_geometry():
    try:
        info = plsc.get_sparse_core_info()
        return info.num_cores, info.num_subcores, info.num_lanes
    except Exception:
        return 2, 16, 16


@functools.lru_cache(maxsize=None)
def _build(P, NELEM, TAB, C, interpret=False):
    NC, NS, L = _sc_geometry()
    NW = NC * NS
    per_w = P // NW
    n_chunks = per_w // C
    n_vec = C // L
    mesh = plsc.VectorSubcoreMesh(core_axis_name="c", subcore_axis_name="s")

    @functools.partial(
        pl.kernel,
        out_type=jax.ShapeDtypeStruct((P,), jnp.float32),
        mesh=mesh,
        scratch_types=[
            pltpu.VMEM((TAB,), jnp.float32),
            pltpu.VMEM((L,), jnp.int32),
            pltpu.VMEM((C,), jnp.int32),
            pltpu.VMEM((C,), jnp.int32),
            pltpu.VMEM((C,), jnp.int32),
            pltpu.VMEM((C,), jnp.int32),
            pltpu.VMEM((C,), jnp.float32),
            pltpu.VMEM((C,), jnp.float32),
            pltpu.VMEM((C,), jnp.float32),
            pltpu.VMEM((C,), jnp.float32),
            pltpu.SemaphoreType.DMA,
            pltpu.SemaphoreType.DMA,
            pltpu.SemaphoreType.DMA,
            pltpu.SemaphoreType.DMA,
        ],
        compiler_params=pltpu.CompilerParams(needs_layout_passes=False),
        interpret=interpret,
    )
    def sc_kernel(sp_hbm, d_hbm, tab_hbm, sel_hbm, out_hbm,
                  tab_v, sel_v, s0_a, s0_b, s1_a, s1_b, d_a, d_b, o_a, o_b,
                  sem_in0, sem_in1, sem_out0, sem_out1):
        wid = lax.axis_index("s") * NC + lax.axis_index("c")
        base = pl.multiple_of(wid * per_w, 8)
        s0_v = (s0_a, s0_b)
        s1_v = (s1_a, s1_b)
        d_v = (d_a, d_b)
        out_v = (o_a, o_b)
        sem_in = (sem_in0, sem_in1)
        sem_out = (sem_out0, sem_out1)

        def start_in(j):
            b = j % 2
            off = pl.multiple_of(base + j * C, 8)
            return [
                pltpu.async_copy(sp_hbm.at[pl.ds(off, C)], s0_v[b], sem_in[b]),
                pltpu.async_copy(sp_hbm.at[pl.ds(P + off, C)], s1_v[b], sem_in[b]),
                pltpu.async_copy(d_hbm.at[pl.ds(off, C)], d_v[b], sem_in[b]),
            ]

        in_flight = {0: start_in(0)}
        pltpu.sync_copy(tab_hbm, tab_v)
        pltpu.sync_copy(sel_hbm, sel_v)
        is6 = sel_v[...] > 0

        out_flight = {}
        for j in range(n_chunks):
            b = j % 2
            if j + 1 < n_chunks:
                in_flight[j + 1] = start_in(j + 1)
            for h in in_flight.pop(j):
                h.wait()
            if j - 2 in out_flight:
                out_flight.pop(j - 2).wait()

            def vec_body(i, carry2):
                s0 = s0_v[b][pl.ds(i * L, L)]
                s1 = s1_v[b][pl.ds(i * L, L)]
                dd = d_v[b][pl.ds(i * L, L)]
                idx = s0 * NELEM + s1
                g = plsc.load_gather(tab_v, [idx])
                r = 1.0 / (1.0 + dd * g)
                t = dd * r
                t2 = t * t
                t4 = t2 * t2
                t6 = t4 * t2
                t8 = t4 * t4
                r2 = r * r
                r4 = r2 * r2
                r8 = r4 * r4
                ts = jnp.where(is6, t6, t8)
                out_v[b][pl.ds(i * L, L)] = ts * r8
                return carry2

            lax.fori_loop(0, n_vec, vec_body, 0, unroll=5)
            off = pl.multiple_of(base + j * C, 8)
            out_flight[j] = pltpu.async_copy(
                out_v[b], out_hbm.at[pl.ds(off, C)], sem_out[b])
        for h in out_flight.values():
            h.wait()

    return sc_kernel


def kernel(species12, distances, order, cutoff_radii, sr6, sr8):
    P = distances.shape[0]
    NELEM = cutoff_radii.shape[0]
    is6 = order == 6
    s = jnp.where(is6, sr6, sr8)
    itab = (jnp.float32(6.0) / (s * cutoff_radii)).reshape(-1)
    TAB = itab.shape[0]
    sel = jnp.where(is6, jnp.int32(1), jnp.int32(0))
    sel_vec = jnp.full((16,), sel, dtype=jnp.int32)
    sc_kernel = _build(P, NELEM, TAB, 10000)
    return sc_kernel(species12.reshape(-1), distances, itab, sel_vec)


# speedup vs baseline: 253.5254x; 2.0360x over previous
import functools

import jax
import jax.numpy as jnp
from jax import lax
from jax.experimental import pallas as pl
from jax.experimental.pallas import tpu as pltpu
from jax.experimental.pallas import tpu_sc as plsc


def _sc_geometry():
    try:
        info = plsc.get_sparse_core_info()
        return info.num_cores, info.num_subcores, info.num_lanes
    except Exception:
        return 2, 16, 16


@functools.lru_cache(maxsize=None)
def _build(P, NELEM, TAB, C, interpret=False):
    NC, NS, L = _sc_geometry()
    NW = NC * NS
    per_w = P // NW
    n_chunks = per_w // C
    n_vec = C // L
    mesh = plsc.VectorSubcoreMesh(core_axis_name="c", subcore_axis_name="s")

    @functools.partial(
        pl.kernel,
        out_type=jax.ShapeDtypeStruct((P,), jnp.float32),
        mesh=mesh,
        scratch_types=[
            pltpu.VMEM((TAB,), jnp.float32),
            pltpu.VMEM((L,), jnp.int32),
            pltpu.VMEM((C,), jnp.int32),
            pltpu.VMEM((C,), jnp.int32),
            pltpu.VMEM((C,), jnp.int32),
            pltpu.VMEM((C,), jnp.int32),
            pltpu.VMEM((C,), jnp.float32),
            pltpu.VMEM((C,), jnp.float32),
            pltpu.VMEM((C,), jnp.float32),
            pltpu.VMEM((C,), jnp.float32),
            pltpu.SemaphoreType.DMA,
            pltpu.SemaphoreType.DMA,
            pltpu.SemaphoreType.DMA,
            pltpu.SemaphoreType.DMA,
        ],
        compiler_params=pltpu.CompilerParams(needs_layout_passes=False),
        interpret=interpret,
    )
    def sc_kernel(sp_hbm, d_hbm, tab_hbm, sel_hbm, out_hbm,
                  tab_v, sel_v, s0_a, s0_b, s1_a, s1_b, d_a, d_b, o_a, o_b,
                  sem_in0, sem_in1, sem_out0, sem_out1):
        wid = lax.axis_index("s") * NC + lax.axis_index("c")
        base = pl.multiple_of(wid * per_w, 8)
        s0_v = (s0_a, s0_b)
        s1_v = (s1_a, s1_b)
        d_v = (d_a, d_b)
        out_v = (o_a, o_b)
        sem_in = (sem_in0, sem_in1)
        sem_out = (sem_out0, sem_out1)

        def start_in(j):
            b = j % 2
            off = pl.multiple_of(base + j * C, 8)
            return [
                pltpu.async_copy(sp_hbm.at[pl.ds(off, C)], s0_v[b], sem_in[b]),
                pltpu.async_copy(sp_hbm.at[pl.ds(P + off, C)], s1_v[b], sem_in[b]),
                pltpu.async_copy(d_hbm.at[pl.ds(off, C)], d_v[b], sem_in[b]),
            ]

        in_flight = {0: start_in(0)}
        pltpu.sync_copy(tab_hbm, tab_v)
        pltpu.sync_copy(sel_hbm, sel_v)
        is6 = sel_v[...] > 0

        out_flight = {}
        for j in range(n_chunks):
            b = j % 2
            if j + 1 < n_chunks:
                in_flight[j + 1] = start_in(j + 1)
            for h in in_flight.pop(j):
                h.wait()
            if j - 2 in out_flight:
                out_flight.pop(j - 2).wait()

            @plsc.parallel_loop(0, n_vec, unroll=5)
            def vec_body(i):
                s0 = s0_v[b][pl.ds(i * L, L)]
                s1 = s1_v[b][pl.ds(i * L, L)]
                dd = d_v[b][pl.ds(i * L, L)]
                idx = s0 * NELEM + s1
                g = plsc.load_gather(tab_v, [idx])
                r = 1.0 / (1.0 + dd * g)
                t = dd * r
                t2 = t * t
                t4 = t2 * t2
                t6 = t4 * t2
                t8 = t4 * t4
                r2 = r * r
                r4 = r2 * r2
                r8 = r4 * r4
                ts = jnp.where(is6, t6, t8)
                out_v[b][pl.ds(i * L, L)] = ts * r8

            off = pl.multiple_of(base + j * C, 8)
            out_flight[j] = pltpu.async_copy(
                out_v[b], out_hbm.at[pl.ds(off, C)], sem_out[b])
        for h in out_flight.values():
            h.wait()

    return sc_kernel


def kernel(species12, distances, order, cutoff_radii, sr6, sr8):
    P = distances.shape[0]
    NELEM = cutoff_radii.shape[0]
    is6 = order == 6
    s = jnp.where(is6, sr6, sr8)
    itab = (jnp.float32(6.0) / (s * cutoff_radii)).reshape(-1)
    TAB = itab.shape[0]
    sel = jnp.where(is6, jnp.int32(1), jnp.int32(0))
    sel_vec = jnp.full((16,), sel, dtype=jnp.int32)
    sc_kernel = _build(P, NELEM, TAB, 10000)
    return sc_kernel(species12.reshape(-1), distances, itab, sel_vec)


# speedup vs baseline: 433.9170x; 1.7115x over previous
import functools

import jax
import jax.numpy as jnp
from jax import lax
from jax.experimental import pallas as pl
from jax.experimental.pallas import tpu as pltpu
from jax.experimental.pallas import tpu_sc as plsc


def _sc_geometry():
    try:
        info = plsc.get_sparse_core_info()
        return info.num_cores, info.num_subcores, info.num_lanes
    except Exception:
        return 2, 16, 16


@functools.lru_cache(maxsize=None)
def _build(P, NELEM, TAB, C):
    NC, NS, L = _sc_geometry()
    NW = NC * NS
    n_total = P // C
    slots = -(-n_total // NW)
    n_vec = C // L
    mesh = plsc.VectorSubcoreMesh(core_axis_name="c", subcore_axis_name="s")

    @functools.partial(
        pl.kernel,
        out_type=jax.ShapeDtypeStruct((P,), jnp.float32),
        mesh=mesh,
        scratch_types=[
            pltpu.VMEM((TAB,), jnp.float32),
            pltpu.VMEM((L,), jnp.int32),
            pltpu.VMEM((2, C), jnp.int32),
            pltpu.VMEM((2, C), jnp.int32),
            pltpu.VMEM((C,), jnp.float32),
            pltpu.VMEM((C,), jnp.float32),
            pltpu.VMEM((C,), jnp.float32),
            pltpu.VMEM((C,), jnp.float32),
            pltpu.SemaphoreType.DMA,
            pltpu.SemaphoreType.DMA,
            pltpu.SemaphoreType.DMA,
            pltpu.SemaphoreType.DMA,
        ],
        compiler_params=pltpu.CompilerParams(needs_layout_passes=False),
    )
    def sc_kernel(sp_hbm, d_hbm, tab_hbm, sel_hbm, out_hbm,
                  tab_v, sel_v, sp_a, sp_b, d_a, d_b, o_a, o_b,
                  sem_in0, sem_in1, sem_out0, sem_out1):
        wid = lax.axis_index("s") * NC + lax.axis_index("c")
        sp_v = (sp_a, sp_b)
        d_v = (d_a, d_b)
        out_v = (o_a, o_b)
        sem_in = (sem_in0, sem_in1)
        sem_out = (sem_out0, sem_out1)

        def cid(m):
            return wid + m * NW

        def in_copies(m):
            b = m % 2
            off = pl.multiple_of(cid(m) * C, 128)
            return [
                pltpu.make_async_copy(
                    sp_hbm.at[:, pl.ds(off, C)], sp_v[b], sem_in[b]),
                pltpu.make_async_copy(
                    d_hbm.at[pl.ds(off, C)], d_v[b], sem_in[b]),
            ]

        def out_copy(m):
            b = m % 2
            off = pl.multiple_of(cid(m) * C, 128)
            return pltpu.make_async_copy(
                out_v[b], out_hbm.at[pl.ds(off, C)], sem_out[b])

        def guarded(m, fn):
            if (m + 1) * NW <= n_total:
                fn()
            else:
                pl.when(cid(m) < n_total)(fn)

        def start_in(m):
            def _go():
                for c in in_copies(m):
                    c.start()
            return _go

        guarded(0, start_in(0))
        pltpu.sync_copy(tab_hbm, tab_v)
        pltpu.sync_copy(sel_hbm, sel_v)
        is6 = sel_v[...] > 0

        for m in range(slots):
            b = m % 2
            if m + 1 < slots:
                guarded(m + 1, start_in(m + 1))

            def slot_body(m=m, b=b):
                for c in in_copies(m):
                    c.wait()
                if m >= 2:
                    out_copy(m - 2).wait()

                @plsc.parallel_loop(0, n_vec, unroll=5)
                def vec_body(i):
                    s0 = sp_v[b][0, pl.ds(i * L, L)]
                    s1 = sp_v[b][1, pl.ds(i * L, L)]
                    dd = d_v[b][pl.ds(i * L, L)]
                    idx = s0 * NELEM + s1
                    g = plsc.load_gather(tab_v, [idx])
                    r = 1.0 / (1.0 + dd * g)
                    t = dd * r
                    t2 = t * t
                    t4 = t2 * t2
                    t6 = t4 * t2
                    t8 = t4 * t4
                    r2 = r * r
                    r4 = r2 * r2
                    r8 = r4 * r4
                    ts = jnp.where(is6, t6, t8)
                    out_v[b][pl.ds(i * L, L)] = ts * r8

                out_copy(m).start()

            guarded(m, slot_body)

        for m in range(max(slots - 2, 0), slots):
            def drain(m=m):
                out_copy(m).wait()
            guarded(m, drain)

    return sc_kernel


def kernel(species12, distances, order, cutoff_radii, sr6, sr8):
    P = distances.shape[0]
    NELEM = cutoff_radii.shape[0]
    is6 = order == 6
    s = jnp.where(is6, sr6, sr8)
    itab = (jnp.float32(6.0) / (s * cutoff_radii)).reshape(-1)
    TAB = itab.shape[0]
    sel = jnp.where(is6, jnp.int32(1), jnp.int32(0))
    sel_vec = jnp.full((16,), sel, dtype=jnp.int32)
    sc_kernel = _build(P, NELEM, TAB, 6400)
    return sc_kernel(species12, distances, itab, sel_vec)


# speedup vs baseline: 442.3054x; 1.0193x over previous
import functools

import jax
import jax.numpy as jnp
from jax import lax
from jax.experimental import pallas as pl
from jax.experimental.pallas import tpu as pltpu
from jax.experimental.pallas import tpu_sc as plsc


def _sc_geometry():
    try:
        info = plsc.get_sparse_core_info()
        return info.num_cores, info.num_subcores, info.num_lanes
    except Exception:
        return 2, 16, 16


@functools.lru_cache(maxsize=None)
def _build(P, NELEM, TAB, C):
    NC, NS, L = _sc_geometry()
    NW = NC * NS
    n_total = P // C
    slots = -(-n_total // NW)
    n_vec = C // L
    mesh = plsc.VectorSubcoreMesh(core_axis_name="c", subcore_axis_name="s")

    @functools.partial(
        pl.kernel,
        out_type=jax.ShapeDtypeStruct((P,), jnp.float32),
        mesh=mesh,
        scratch_types=[
            pltpu.VMEM((TAB,), jnp.float32),
            pltpu.VMEM((L,), jnp.int32),
            pltpu.VMEM((2, C), jnp.int32),
            pltpu.VMEM((2, C), jnp.int32),
            pltpu.VMEM((C,), jnp.float32),
            pltpu.VMEM((C,), jnp.float32),
            pltpu.VMEM((C,), jnp.float32),
            pltpu.VMEM((C,), jnp.float32),
            pltpu.SemaphoreType.DMA,
            pltpu.SemaphoreType.DMA,
            pltpu.SemaphoreType.DMA,
            pltpu.SemaphoreType.DMA,
        ],
        compiler_params=pltpu.CompilerParams(needs_layout_passes=False),
    )
    def sc_kernel(sp_hbm, d_hbm, tab_hbm, sel_hbm, out_hbm,
                  tab_v, sel_v, sp_a, sp_b, d_a, d_b, o_a, o_b,
                  sem_in0, sem_in1, sem_out0, sem_out1):
        wid = lax.axis_index("s") * NC + lax.axis_index("c")
        sp_v = (sp_a, sp_b)
        d_v = (d_a, d_b)
        out_v = (o_a, o_b)
        sem_in = (sem_in0, sem_in1)
        sem_out = (sem_out0, sem_out1)

        def cid(m):
            return wid + m * NW

        def in_copies(m):
            b = m % 2
            off = pl.multiple_of(cid(m) * C, 128)
            return [
                pltpu.make_async_copy(
                    sp_hbm.at[:, pl.ds(off, C)], sp_v[b], sem_in[b]),
                pltpu.make_async_copy(
                    d_hbm.at[pl.ds(off, C)], d_v[b], sem_in[b]),
            ]

        def out_copy(m):
            b = m % 2
            off = pl.multiple_of(cid(m) * C, 128)
            return pltpu.make_async_copy(
                out_v[b], out_hbm.at[pl.ds(off, C)], sem_out[b])

        def guarded(m, fn):
            if (m + 1) * NW <= n_total:
                fn()
            else:
                pl.when(cid(m) < n_total)(fn)

        def start_in(m):
            def _go():
                for c in in_copies(m):
                    c.start()
            return _go

        guarded(0, start_in(0))
        pltpu.sync_copy(tab_hbm, tab_v)
        pltpu.sync_copy(sel_hbm, sel_v)
        is6 = sel_v[...] > 0

        for m in range(slots):
            b = m % 2
            if m + 1 < slots:
                guarded(m + 1, start_in(m + 1))

            def slot_body(m=m, b=b):
                for c in in_copies(m):
                    c.wait()
                if m >= 2:
                    out_copy(m - 2).wait()

                @plsc.parallel_loop(0, n_vec, unroll=8)
                def vec_body(i):
                    s0 = sp_v[b][0, pl.ds(i * L, L)]
                    s1 = sp_v[b][1, pl.ds(i * L, L)]
                    dd = d_v[b][pl.ds(i * L, L)]
                    idx = s0 * NELEM + s1
                    g = plsc.load_gather(tab_v, [idx])
                    r = 1.0 / (1.0 + dd * g)
                    t = dd * r
                    t2 = t * t
                    t4 = t2 * t2
                    t6 = t4 * t2
                    t8 = t4 * t4
                    r2 = r * r
                    r4 = r2 * r2
                    r8 = r4 * r4
                    ts = jnp.where(is6, t6, t8)
                    out_v[b][pl.ds(i * L, L)] = ts * r8

                out_copy(m).start()

            guarded(m, slot_body)

        for m in range(max(slots - 2, 0), slots):
            def drain(m=m):
                out_copy(m).wait()
            guarded(m, drain)

    return sc_kernel


def kernel(species12, distances, order, cutoff_radii, sr6, sr8):
    P = distances.shape[0]
    NELEM = cutoff_radii.shape[0]
    is6 = order == 6
    s = jnp.where(is6, sr6, sr8)
    itab = (jnp.float32(6.0) / (s * cutoff_radii)).reshape(-1)
    TAB = itab.shape[0]
    sel = jnp.where(is6, jnp.int32(1), jnp.int32(0))
    sel_vec = jnp.full((16,), sel, dtype=jnp.int32)
    sc_kernel = _build(P, NELEM, TAB, 12800)
    return sc_kernel(species12, distances, itab, sel_vec)
